# Initial kernel scaffold; baseline (speedup 1.0000x reference)
import jax
import jax.numpy as jnp
from jax.experimental import pallas as pl


def kernel(expected, drift):
    raise NotImplementedError("write your pallas kernel here")


# speedup vs baseline: 1.4922x; 1.4922x over previous
import jax
import jax.numpy as jnp
from jax.experimental import pallas as pl

S = 32768
A = 1024
N = 256
BLOCK = 2048
GRID = S // BLOCK


def _body(x_ref, d_ref, o_ref):
    i = pl.program_id(0)

    @pl.when(i < GRID - 1)
    def _copy():
        o_ref[...] = x_ref[...]

    @pl.when(i == GRID - 1)
    def _fixup():
        o_ref[0:BLOCK - N, :] = x_ref[0:BLOCK - N, :]
        o_ref[BLOCK - N:BLOCK, :] = (
            x_ref[BLOCK - 2 * N:BLOCK - N, :] + d_ref[...]
        )


def kernel(expected, drift):
    drift2d = drift.reshape(N, 1)
    return pl.pallas_call(
        _body,
        grid=(GRID,),
        in_specs=[
            pl.BlockSpec((BLOCK, A), lambda i: (i, 0)),
            pl.BlockSpec((N, 1), lambda i: (0, 0)),
        ],
        out_specs=pl.BlockSpec((BLOCK, A), lambda i: (i, 0)),
        out_shape=jax.ShapeDtypeStruct((S, A), expected.dtype),
    )(expected, drift2d)
